# Initial kernel scaffold; baseline (speedup 1.0000x reference)
import functools

import jax
import jax.numpy as jnp
from jax import lax
from jax.experimental import pallas as pl
from jax.experimental.pallas import tpu as pltpu
from jax.experimental.pallas import tpu_sc as plsc

_N = 100000
_D = 32
_HALF = _N // 2
_NSUB = 16
_ACC_ROWS = 51200
_K = 80
_NBUF = 4
_BN = 10000


def _sc_scatter_add(feats, src, dst):
    E = src.shape[0]
    per_sub = E // _NSUB
    n_outer = per_sub // (_K * _NBUF)
    mesh = plsc.VectorSubcoreMesh(core_axis_name="c", subcore_axis_name="s")

    scratch = [pltpu.VMEM_SHARED((_ACC_ROWS, _D), jnp.float32)]
    scratch += [pltpu.VMEM((_K,), jnp.int32) for _ in range(_NBUF)]
    scratch += [pltpu.VMEM((_K,), jnp.int32) for _ in range(_NBUF)]
    scratch += [pltpu.VMEM((_K,), jnp.int32) for _ in range(_NBUF)]
    scratch += [pltpu.VMEM((_K, _D), jnp.float32) for _ in range(_NBUF)]
    scratch += [pltpu.SemaphoreType.DMA for _ in range(3 * _NBUF)]

    @functools.partial(
        pl.kernel,
        out_type=jax.ShapeDtypeStruct((_N, _D), jnp.float32),
        mesh=mesh,
        scratch_types=scratch,
    )
    def k(feats_hbm, src_hbm, dst_hbm, out_hbm, acc, *sc):
        srcv = sc[0:_NBUF]
        dstraw = sc[_NBUF:2 * _NBUF]
        dstloc = sc[2 * _NBUF:3 * _NBUF]
        rows = sc[3 * _NBUF:4 * _NBUF]
        sem_i = sc[4 * _NBUF:5 * _NBUF]
        sem_g = sc[5 * _NBUF:6 * _NBUF]
        sem_s = sc[6 * _NBUF:7 * _NBUF]
        c = lax.axis_index("c")
        s = lax.axis_index("s")
        cbase = c * _HALF
        zero16 = jnp.zeros((16,), jnp.float32)

        @pl.loop(0, _K)
        def _(r):
            rows[0][r, pl.ds(0, 16)] = zero16
            rows[0][r, pl.ds(16, 16)] = zero16

        zrows = _ACC_ROWS // _NSUB

        @pl.loop(0, zrows // _K)
        def _(t):
            off = pl.multiple_of(s * zrows + t * _K, 8)
            pltpu.sync_copy(rows[0], acc.at[pl.ds(off, _K)])

        plsc.subcore_barrier()
        base_sub = s * per_sub

        @pl.loop(0, n_outer)
        def _(t):
            descs_i = []
            for b in range(_NBUF):
                base = pl.multiple_of(base_sub + (t * _NBUF + b) * _K, 8)
                d1 = pltpu.async_copy(src_hbm.at[pl.ds(base, _K)], srcv[b], sem_i[b])
                d2 = pltpu.async_copy(dst_hbm.at[pl.ds(base, _K)], dstraw[b], sem_i[b])
                descs_i.append((d1, d2))
            descs_g = []
            for b in range(_NBUF):
                d1, d2 = descs_i[b]
                d1.wait()
                d2.wait()
                descs_g.append(
                    pltpu.async_copy(feats_hbm.at[srcv[b]], rows[b], sem_g[b]))
            for b in range(_NBUF):
                @pl.loop(0, _K, step=16)
                def _(i, _b=b):
                    i = pl.multiple_of(i, 8)
                    d = dstraw[_b][pl.ds(i, 16)]
                    l = d - cbase
                    inb = (l >= 0) & (l < _HALF)
                    dstloc[_b][pl.ds(i, 16)] = jnp.where(inb, l, _HALF)
            descs_s = []
            for b in range(_NBUF):
                descs_g[b].wait()
                descs_s.append(
                    pltpu.async_copy(rows[b], acc.at[dstloc[b]], sem_s[b], add=True))
            for b in range(_NBUF):
                descs_s[b].wait()

        plsc.subcore_barrier()
        wrows = _HALF // _NSUB
        woff = pl.multiple_of(s * wrows, 8)
        goff = pl.multiple_of(cbase + s * wrows, 8)
        pltpu.sync_copy(acc.at[pl.ds(woff, wrows)], out_hbm.at[pl.ds(goff, wrows)])

    return k(feats, src, dst)


def _expand(x, W1a):
    def body(x_ref, w_ref, y_ref):
        y_ref[...] = x_ref[...] * w_ref[...]

    return pl.pallas_call(
        body,
        grid=(_N // _BN,),
        in_specs=[
            pl.BlockSpec((_BN, 1), lambda i: (i, 0)),
            pl.BlockSpec((1, _D), lambda i: (0, 0)),
        ],
        out_specs=pl.BlockSpec((_BN, _D), lambda i: (i, 0)),
        out_shape=jax.ShapeDtypeStruct((_N, _D), jnp.float32),
    )(x, W1a)


def _mlp(a, b, Wa, ba, Wb, bb):
    grid = _N // _BN

    def body(a_ref, b_ref, wa_ref, ba_ref, wb_ref, bb_ref, v_ref, sums_ref, acc_ref):
        i = pl.program_id(0)

        @pl.when(i == 0)
        def _():
            acc_ref[...] = jnp.zeros_like(acc_ref)

        z = a_ref[...] + b_ref[...]
        t = jnp.maximum(
            jnp.dot(z, wa_ref[...], preferred_element_type=jnp.float32) + ba_ref[...],
            0.0)
        u = jnp.dot(t, wb_ref[...], preferred_element_type=jnp.float32) + bb_ref[...]
        v = jnp.maximum(u, 0.0)
        v_ref[...] = v
        acc_ref[0:1, :] += jnp.sum(v, axis=0, keepdims=True)
        acc_ref[1:2, :] += jnp.sum(v * v, axis=0, keepdims=True)

        @pl.when(i == grid - 1)
        def _():
            sums_ref[...] = acc_ref[...]

    return pl.pallas_call(
        body,
        grid=(grid,),
        in_specs=[
            pl.BlockSpec((_BN, _D), lambda i: (i, 0)),
            pl.BlockSpec((_BN, _D), lambda i: (i, 0)),
            pl.BlockSpec((_D, _D), lambda i: (0, 0)),
            pl.BlockSpec((1, _D), lambda i: (0, 0)),
            pl.BlockSpec((_D, _D), lambda i: (0, 0)),
            pl.BlockSpec((1, _D), lambda i: (0, 0)),
        ],
        out_specs=[
            pl.BlockSpec((_BN, _D), lambda i: (i, 0)),
            pl.BlockSpec((8, _D), lambda i: (0, 0)),
        ],
        out_shape=[
            jax.ShapeDtypeStruct((_N, _D), jnp.float32),
            jax.ShapeDtypeStruct((8, _D), jnp.float32),
        ],
        scratch_shapes=[pltpu.VMEM((8, _D), jnp.float32)],
    )(a, b, Wa, ba, Wb, bb)


def _norm(v, sums, g, be):
    def body(v_ref, s_ref, g_ref, be_ref, h_ref):
        mu = s_ref[0:1, :] * (1.0 / _N)
        var = s_ref[1:2, :] * (1.0 / _N) - mu * mu
        scale = lax.rsqrt(var + 1e-5) * g_ref[...]
        h_ref[...] = (v_ref[...] - mu) * scale + be_ref[...]

    return pl.pallas_call(
        body,
        grid=(_N // _BN,),
        in_specs=[
            pl.BlockSpec((_BN, _D), lambda i: (i, 0)),
            pl.BlockSpec((8, _D), lambda i: (0, 0)),
            pl.BlockSpec((1, _D), lambda i: (0, 0)),
            pl.BlockSpec((1, _D), lambda i: (0, 0)),
        ],
        out_specs=pl.BlockSpec((_BN, _D), lambda i: (i, 0)),
        out_shape=jax.ShapeDtypeStruct((_N, _D), jnp.float32),
    )(v, sums, g, be)


def _final(v, sums, g, be, fc1_W, fc1_b, fc2_W, fc2_b):
    nc = fc2_W.shape[1]

    def body(v_ref, s_ref, g_ref, be_ref, w1_ref, b1_ref, w2_ref, b2_ref, o_ref):
        mu = s_ref[0:1, :] * (1.0 / _N)
        var = s_ref[1:2, :] * (1.0 / _N) - mu * mu
        scale = lax.rsqrt(var + 1e-5) * g_ref[...]
        h = (v_ref[...] - mu) * scale + be_ref[...]
        r = jnp.maximum(
            jnp.dot(h, w1_ref[...], preferred_element_type=jnp.float32) + b1_ref[...],
            0.0)
        o = jnp.dot(r, w2_ref[...], preferred_element_type=jnp.float32) + b2_ref[...]
        m = jnp.max(o, axis=-1, keepdims=True)
        lse = m + jnp.log(jnp.sum(jnp.exp(o - m), axis=-1, keepdims=True))
        o_ref[...] = o - lse

    return pl.pallas_call(
        body,
        grid=(_N // _BN,),
        in_specs=[
            pl.BlockSpec((_BN, _D), lambda i: (i, 0)),
            pl.BlockSpec((8, _D), lambda i: (0, 0)),
            pl.BlockSpec((1, _D), lambda i: (0, 0)),
            pl.BlockSpec((1, _D), lambda i: (0, 0)),
            pl.BlockSpec((_D, _D), lambda i: (0, 0)),
            pl.BlockSpec((1, _D), lambda i: (0, 0)),
            pl.BlockSpec((_D, nc), lambda i: (0, 0)),
            pl.BlockSpec((1, nc), lambda i: (0, 0)),
        ],
        out_specs=pl.BlockSpec((_BN, nc), lambda i: (i, 0)),
        out_shape=jax.ShapeDtypeStruct((_N, nc), jnp.float32),
    )(v, sums, g, be, fc1_W, fc1_b, fc2_W, fc2_b)


def kernel(x, edge_index, W1a, b1a, W1b, b1b, g1, be1, W2a, b2a, W2b, b2b, g2,
           be2, W3a, b3a, W3b, b3b, g3, be3, W4a, b4a, W4b, b4b, g4, be4, W5a,
           b5a, W5b, b5b, g5, be5, fc1_W, fc1_b, fc2_W, fc2_b):
    src = edge_index[0]
    dst = edge_index[1]
    eye = jnp.eye(_D, dtype=jnp.float32)

    y1 = _expand(x, W1a)
    aggy = _sc_scatter_add(y1, src, dst)
    v, sums = _mlp(y1, aggy, eye, b1a.reshape(1, _D), W1b, b1b.reshape(1, _D))
    h = _norm(v, sums, g1.reshape(1, _D), be1.reshape(1, _D))

    layers = [
        (W2a, b2a, W2b, b2b, g2, be2),
        (W3a, b3a, W3b, b3b, g3, be3),
        (W4a, b4a, W4b, b4b, g4, be4),
        (W5a, b5a, W5b, b5b, g5, be5),
    ]
    for li, (Wa, ba, Wb, bb, g, be) in enumerate(layers):
        agg = _sc_scatter_add(h, src, dst)
        v, sums = _mlp(h, agg, Wa, ba.reshape(1, _D), Wb, bb.reshape(1, _D))
        if li < len(layers) - 1:
            h = _norm(v, sums, g.reshape(1, _D), be.reshape(1, _D))

    return _final(v, sums, g5.reshape(1, _D), be5.reshape(1, _D), fc1_W,
                  fc1_b.reshape(1, _D), fc2_W, fc2_b.reshape(1, 2))


# speedup vs baseline: 10.3999x; 10.3999x over previous
import functools

import jax
import jax.numpy as jnp
from jax import lax
from jax.experimental import pallas as pl
from jax.experimental.pallas import tpu as pltpu
from jax.experimental.pallas import tpu_sc as plsc

_N = 100000
_D = 32
_SPLIT = 50048
_NSUB = 16
_ACC_ROWS = 51200
_TRASH = 51000
_K = 80
_NBUF = 4
_BN = 10000


def _sc_scatter_add(feats, src, dst):
    E = src.shape[0]
    per_sub = E // _NSUB
    n_outer = per_sub // (_K * _NBUF)
    mesh = plsc.VectorSubcoreMesh(core_axis_name="c", subcore_axis_name="s")

    scratch = [pltpu.VMEM_SHARED((_ACC_ROWS, _D), jnp.float32)]
    scratch += [pltpu.VMEM((_K,), jnp.int32) for _ in range(_NBUF)]
    scratch += [pltpu.VMEM((_K,), jnp.int32) for _ in range(_NBUF)]
    scratch += [pltpu.VMEM((_K,), jnp.int32) for _ in range(_NBUF)]
    scratch += [pltpu.VMEM((_K, _D), jnp.float32) for _ in range(_NBUF)]
    scratch += [pltpu.SemaphoreType.DMA for _ in range(3 * _NBUF)]

    @functools.partial(
        pl.kernel,
        out_type=jax.ShapeDtypeStruct((_N, _D), jnp.float32),
        mesh=mesh,
        scratch_types=scratch,
        compiler_params=pltpu.CompilerParams(use_tc_tiling_on_sc=False),
    )
    def k(feats_hbm, src_hbm, dst_hbm, out_hbm, acc, *sc):
        srcv = sc[0:_NBUF]
        dstraw = sc[_NBUF:2 * _NBUF]
        dstloc = sc[2 * _NBUF:3 * _NBUF]
        rows = sc[3 * _NBUF:4 * _NBUF]
        sem_i = sc[4 * _NBUF:5 * _NBUF]
        sem_g = sc[5 * _NBUF:6 * _NBUF]
        sem_s = sc[6 * _NBUF:7 * _NBUF]
        c = lax.axis_index("c")
        s = lax.axis_index("s")
        cbase = c * _SPLIT
        csize = _SPLIT - c * (2 * _SPLIT - _N)
        zero16 = jnp.zeros((16,), jnp.float32)

        @pl.loop(0, _K)
        def _(r):
            rows[0][r, pl.ds(0, 16)] = zero16
            rows[0][r, pl.ds(16, 16)] = zero16

        zrows = _ACC_ROWS // _NSUB

        @pl.loop(0, zrows // _K)
        def _(t):
            off = pl.multiple_of(s * zrows + t * _K, 8)
            pltpu.sync_copy(rows[0], acc.at[pl.ds(off, _K)])

        plsc.subcore_barrier()
        base_sub = s * per_sub

        @pl.loop(0, n_outer)
        def _(t):
            descs_i = []
            for b in range(_NBUF):
                base = pl.multiple_of(base_sub + (t * _NBUF + b) * _K, 8)
                d1 = pltpu.async_copy(src_hbm.at[pl.ds(base, _K)], srcv[b], sem_i[b])
                d2 = pltpu.async_copy(dst_hbm.at[pl.ds(base, _K)], dstraw[b], sem_i[b])
                descs_i.append((d1, d2))
            descs_g = []
            for b in range(_NBUF):
                d1, d2 = descs_i[b]
                d1.wait()
                d2.wait()
                descs_g.append(
                    pltpu.async_copy(feats_hbm.at[srcv[b]], rows[b], sem_g[b]))
            for b in range(_NBUF):
                @pl.loop(0, _K, step=16)
                def _(i, _b=b):
                    i = pl.multiple_of(i, 8)
                    d = dstraw[_b][pl.ds(i, 16)]
                    l = d - cbase
                    inb = (l >= 0) & (l < csize)
                    dstloc[_b][pl.ds(i, 16)] = jnp.where(inb, l, _TRASH)
            descs_s = []
            for b in range(_NBUF):
                descs_g[b].wait()
                descs_s.append(
                    pltpu.async_copy(rows[b], acc.at[dstloc[b]], sem_s[b], add=True))
            for b in range(_NBUF):
                descs_s[b].wait()

        plsc.subcore_barrier()
        full = 3200

        @pl.when(s < _NSUB - 1)
        def _():
            woff = pl.multiple_of(s * full, 8)
            goff = pl.multiple_of(cbase + s * full, 8)
            pltpu.sync_copy(acc.at[pl.ds(woff, full)], out_hbm.at[pl.ds(goff, full)])

        tail_off = (_NSUB - 1) * full

        @pl.when((s == _NSUB - 1) & (c == 0))
        def _():
            pltpu.sync_copy(acc.at[pl.ds(tail_off, _SPLIT - tail_off)],
                            out_hbm.at[pl.ds(tail_off, _SPLIT - tail_off)])

        @pl.when((s == _NSUB - 1) & (c == 1))
        def _():
            pltpu.sync_copy(acc.at[pl.ds(tail_off, _N - _SPLIT - tail_off)],
                            out_hbm.at[pl.ds(_SPLIT + tail_off, _N - _SPLIT - tail_off)])

    return k(feats, src, dst)


def _expand(x, W1a):
    def body(x_ref, w_ref, y_ref):
        y_ref[...] = x_ref[...] * w_ref[...]

    return pl.pallas_call(
        body,
        grid=(_N // _BN,),
        in_specs=[
            pl.BlockSpec((_BN, 1), lambda i: (i, 0)),
            pl.BlockSpec((1, _D), lambda i: (0, 0)),
        ],
        out_specs=pl.BlockSpec((_BN, _D), lambda i: (i, 0)),
        out_shape=jax.ShapeDtypeStruct((_N, _D), jnp.float32),
    )(x, W1a)


def _mlp(a, b, Wa, ba, Wb, bb, first_dot=True):
    grid = _N // _BN

    def body(a_ref, b_ref, wa_ref, ba_ref, wb_ref, bb_ref, v_ref, sums_ref, acc_ref):
        i = pl.program_id(0)

        @pl.when(i == 0)
        def _():
            acc_ref[...] = jnp.zeros_like(acc_ref)

        z = a_ref[...] + b_ref[...]
        if first_dot:
            t = jnp.maximum(
                jnp.dot(z, wa_ref[...], preferred_element_type=jnp.float32)
                + ba_ref[...], 0.0)
        else:
            t = jnp.maximum(z + ba_ref[...], 0.0)
        u = jnp.dot(t, wb_ref[...], preferred_element_type=jnp.float32) + bb_ref[...]
        v = jnp.maximum(u, 0.0)
        v_ref[...] = v
        acc_ref[0:1, :] += jnp.sum(v, axis=0, keepdims=True)
        acc_ref[1:2, :] += jnp.sum(v * v, axis=0, keepdims=True)

        @pl.when(i == grid - 1)
        def _():
            sums_ref[...] = acc_ref[...]

    return pl.pallas_call(
        body,
        grid=(grid,),
        in_specs=[
            pl.BlockSpec((_BN, _D), lambda i: (i, 0)),
            pl.BlockSpec((_BN, _D), lambda i: (i, 0)),
            pl.BlockSpec((_D, _D), lambda i: (0, 0)),
            pl.BlockSpec((1, _D), lambda i: (0, 0)),
            pl.BlockSpec((_D, _D), lambda i: (0, 0)),
            pl.BlockSpec((1, _D), lambda i: (0, 0)),
        ],
        out_specs=[
            pl.BlockSpec((_BN, _D), lambda i: (i, 0)),
            pl.BlockSpec((8, _D), lambda i: (0, 0)),
        ],
        out_shape=[
            jax.ShapeDtypeStruct((_N, _D), jnp.float32),
            jax.ShapeDtypeStruct((8, _D), jnp.float32),
        ],
        scratch_shapes=[pltpu.VMEM((8, _D), jnp.float32)],
    )(a, b, Wa, ba, Wb, bb)


def _norm(v, sums, g, be):
    def body(v_ref, s_ref, g_ref, be_ref, h_ref):
        mu = s_ref[0:1, :] * (1.0 / _N)
        var = s_ref[1:2, :] * (1.0 / _N) - mu * mu
        scale = lax.rsqrt(var + 1e-5) * g_ref[...]
        h_ref[...] = (v_ref[...] - mu) * scale + be_ref[...]

    return pl.pallas_call(
        body,
        grid=(_N // _BN,),
        in_specs=[
            pl.BlockSpec((_BN, _D), lambda i: (i, 0)),
            pl.BlockSpec((8, _D), lambda i: (0, 0)),
            pl.BlockSpec((1, _D), lambda i: (0, 0)),
            pl.BlockSpec((1, _D), lambda i: (0, 0)),
        ],
        out_specs=pl.BlockSpec((_BN, _D), lambda i: (i, 0)),
        out_shape=jax.ShapeDtypeStruct((_N, _D), jnp.float32),
    )(v, sums, g, be)


def _final(v, sums, g, be, fc1_W, fc1_b, fc2_W, fc2_b):
    nc = fc2_W.shape[1]

    def body(v_ref, s_ref, g_ref, be_ref, w1_ref, b1_ref, w2_ref, b2_ref, o_ref):
        mu = s_ref[0:1, :] * (1.0 / _N)
        var = s_ref[1:2, :] * (1.0 / _N) - mu * mu
        scale = lax.rsqrt(var + 1e-5) * g_ref[...]
        h = (v_ref[...] - mu) * scale + be_ref[...]
        r = jnp.maximum(
            jnp.dot(h, w1_ref[...], preferred_element_type=jnp.float32) + b1_ref[...],
            0.0)
        o = jnp.dot(r, w2_ref[...], preferred_element_type=jnp.float32) + b2_ref[...]
        m = jnp.max(o, axis=-1, keepdims=True)
        lse = m + jnp.log(jnp.sum(jnp.exp(o - m), axis=-1, keepdims=True))
        o_ref[...] = o - lse

    return pl.pallas_call(
        body,
        grid=(_N // _BN,),
        in_specs=[
            pl.BlockSpec((_BN, _D), lambda i: (i, 0)),
            pl.BlockSpec((8, _D), lambda i: (0, 0)),
            pl.BlockSpec((1, _D), lambda i: (0, 0)),
            pl.BlockSpec((1, _D), lambda i: (0, 0)),
            pl.BlockSpec((_D, _D), lambda i: (0, 0)),
            pl.BlockSpec((1, _D), lambda i: (0, 0)),
            pl.BlockSpec((_D, nc), lambda i: (0, 0)),
            pl.BlockSpec((1, nc), lambda i: (0, 0)),
        ],
        out_specs=pl.BlockSpec((_BN, nc), lambda i: (i, 0)),
        out_shape=jax.ShapeDtypeStruct((_N, nc), jnp.float32),
    )(v, sums, g, be, fc1_W, fc1_b, fc2_W, fc2_b)


def kernel(x, edge_index, W1a, b1a, W1b, b1b, g1, be1, W2a, b2a, W2b, b2b, g2,
           be2, W3a, b3a, W3b, b3b, g3, be3, W4a, b4a, W4b, b4b, g4, be4, W5a,
           b5a, W5b, b5b, g5, be5, fc1_W, fc1_b, fc2_W, fc2_b):
    src = edge_index[0]
    dst = edge_index[1]
    eye = jnp.eye(_D, dtype=jnp.float32)

    y1 = _expand(x, W1a)
    aggy = _sc_scatter_add(y1, src, dst)
    v, sums = _mlp(y1, aggy, eye, b1a.reshape(1, _D), W1b, b1b.reshape(1, _D),
                   first_dot=False)
    h = _norm(v, sums, g1.reshape(1, _D), be1.reshape(1, _D))

    layers = [
        (W2a, b2a, W2b, b2b, g2, be2),
        (W3a, b3a, W3b, b3b, g3, be3),
        (W4a, b4a, W4b, b4b, g4, be4),
        (W5a, b5a, W5b, b5b, g5, be5),
    ]
    for li, (Wa, ba, Wb, bb, g, be) in enumerate(layers):
        agg = _sc_scatter_add(h, src, dst)
        v, sums = _mlp(h, agg, Wa, ba.reshape(1, _D), Wb, bb.reshape(1, _D))
        if li < len(layers) - 1:
            h = _norm(v, sums, g.reshape(1, _D), be.reshape(1, _D))

    return _final(v, sums, g5.reshape(1, _D), be5.reshape(1, _D), fc1_W,
                  fc1_b.reshape(1, _D), fc2_W, fc2_b.reshape(1, 2))


# speedup vs baseline: 10.4284x; 1.0027x over previous
import functools

import jax
import jax.numpy as jnp
from jax import lax
from jax.experimental import pallas as pl
from jax.experimental.pallas import tpu as pltpu
from jax.experimental.pallas import tpu_sc as plsc

_N = 100000
_D = 32
_SPLIT = 50048
_NSUB = 16
_ACC_ROWS = 51200
_TRASH = 51000
_K = 80
_NBUF = 8
_BN = 10000


def _sc_scatter_add(feats, src, dst):
    E = src.shape[0]
    per_sub = E // _NSUB
    n_outer = per_sub // (_K * _NBUF)
    mesh = plsc.VectorSubcoreMesh(core_axis_name="c", subcore_axis_name="s")

    scratch = [pltpu.VMEM_SHARED((_ACC_ROWS, _D), jnp.float32)]
    scratch += [pltpu.VMEM((_K,), jnp.int32) for _ in range(_NBUF)]
    scratch += [pltpu.VMEM((_K,), jnp.int32) for _ in range(_NBUF)]
    scratch += [pltpu.VMEM((_K,), jnp.int32) for _ in range(_NBUF)]
    scratch += [pltpu.VMEM((_K, _D), jnp.float32) for _ in range(_NBUF)]
    scratch += [pltpu.SemaphoreType.DMA for _ in range(3 * _NBUF)]

    @functools.partial(
        pl.kernel,
        out_type=jax.ShapeDtypeStruct((_N, _D), jnp.float32),
        mesh=mesh,
        scratch_types=scratch,
        compiler_params=pltpu.CompilerParams(use_tc_tiling_on_sc=False),
    )
    def k(feats_hbm, src_hbm, dst_hbm, out_hbm, acc, *sc):
        srcv = sc[0:_NBUF]
        dstraw = sc[_NBUF:2 * _NBUF]
        dstloc = sc[2 * _NBUF:3 * _NBUF]
        rows = sc[3 * _NBUF:4 * _NBUF]
        sem_i = sc[4 * _NBUF:5 * _NBUF]
        sem_g = sc[5 * _NBUF:6 * _NBUF]
        sem_s = sc[6 * _NBUF:7 * _NBUF]
        c = lax.axis_index("c")
        s = lax.axis_index("s")
        cbase = c * _SPLIT
        csize = _SPLIT - c * (2 * _SPLIT - _N)
        zero16 = jnp.zeros((16,), jnp.float32)

        @pl.loop(0, _K)
        def _(r):
            rows[0][r, pl.ds(0, 16)] = zero16
            rows[0][r, pl.ds(16, 16)] = zero16

        zrows = _ACC_ROWS // _NSUB

        @pl.loop(0, zrows // _K)
        def _(t):
            off = pl.multiple_of(s * zrows + t * _K, 8)
            pltpu.sync_copy(rows[0], acc.at[pl.ds(off, _K)])

        plsc.subcore_barrier()
        base_sub = s * per_sub

        @pl.loop(0, n_outer)
        def _(t):
            descs_i = []
            for b in range(_NBUF):
                base = pl.multiple_of(base_sub + (t * _NBUF + b) * _K, 8)
                d1 = pltpu.async_copy(src_hbm.at[pl.ds(base, _K)], srcv[b], sem_i[b])
                d2 = pltpu.async_copy(dst_hbm.at[pl.ds(base, _K)], dstraw[b], sem_i[b])
                descs_i.append((d1, d2))
            descs_g = []
            for b in range(_NBUF):
                d1, d2 = descs_i[b]
                d1.wait()
                d2.wait()
                descs_g.append(
                    pltpu.async_copy(feats_hbm.at[srcv[b]], rows[b], sem_g[b]))
            for b in range(_NBUF):
                @pl.loop(0, _K, step=16)
                def _(i, _b=b):
                    i = pl.multiple_of(i, 8)
                    d = dstraw[_b][pl.ds(i, 16)]
                    l = d - cbase
                    inb = (l >= 0) & (l < csize)
                    dstloc[_b][pl.ds(i, 16)] = jnp.where(inb, l, _TRASH)
            descs_s = []
            for b in range(_NBUF):
                descs_g[b].wait()
                descs_s.append(
                    pltpu.async_copy(rows[b], acc.at[dstloc[b]], sem_s[b], add=True))
            for b in range(_NBUF):
                descs_s[b].wait()

        plsc.subcore_barrier()
        full = 3200

        @pl.when(s < _NSUB - 1)
        def _():
            woff = pl.multiple_of(s * full, 8)
            goff = pl.multiple_of(cbase + s * full, 8)
            pltpu.sync_copy(acc.at[pl.ds(woff, full)], out_hbm.at[pl.ds(goff, full)])

        tail_off = (_NSUB - 1) * full

        @pl.when((s == _NSUB - 1) & (c == 0))
        def _():
            pltpu.sync_copy(acc.at[pl.ds(tail_off, _SPLIT - tail_off)],
                            out_hbm.at[pl.ds(tail_off, _SPLIT - tail_off)])

        @pl.when((s == _NSUB - 1) & (c == 1))
        def _():
            pltpu.sync_copy(acc.at[pl.ds(tail_off, _N - _SPLIT - tail_off)],
                            out_hbm.at[pl.ds(_SPLIT + tail_off, _N - _SPLIT - tail_off)])

    return k(feats, src, dst)


def _expand(x, W1a):
    def body(x_ref, w_ref, y_ref):
        y_ref[...] = x_ref[...] * w_ref[...]

    return pl.pallas_call(
        body,
        grid=(_N // _BN,),
        in_specs=[
            pl.BlockSpec((_BN, 1), lambda i: (i, 0)),
            pl.BlockSpec((1, _D), lambda i: (0, 0)),
        ],
        out_specs=pl.BlockSpec((_BN, _D), lambda i: (i, 0)),
        out_shape=jax.ShapeDtypeStruct((_N, _D), jnp.float32),
    )(x, W1a)


def _mlp(a, b, Wa, ba, Wb, bb, first_dot=True):
    grid = _N // _BN

    def body(a_ref, b_ref, wa_ref, ba_ref, wb_ref, bb_ref, v_ref, sums_ref, acc_ref):
        i = pl.program_id(0)

        @pl.when(i == 0)
        def _():
            acc_ref[...] = jnp.zeros_like(acc_ref)

        z = a_ref[...] + b_ref[...]
        if first_dot:
            t = jnp.maximum(
                jnp.dot(z, wa_ref[...], preferred_element_type=jnp.float32)
                + ba_ref[...], 0.0)
        else:
            t = jnp.maximum(z + ba_ref[...], 0.0)
        u = jnp.dot(t, wb_ref[...], preferred_element_type=jnp.float32) + bb_ref[...]
        v = jnp.maximum(u, 0.0)
        v_ref[...] = v
        acc_ref[0:1, :] += jnp.sum(v, axis=0, keepdims=True)
        acc_ref[1:2, :] += jnp.sum(v * v, axis=0, keepdims=True)

        @pl.when(i == grid - 1)
        def _():
            sums_ref[...] = acc_ref[...]

    return pl.pallas_call(
        body,
        grid=(grid,),
        in_specs=[
            pl.BlockSpec((_BN, _D), lambda i: (i, 0)),
            pl.BlockSpec((_BN, _D), lambda i: (i, 0)),
            pl.BlockSpec((_D, _D), lambda i: (0, 0)),
            pl.BlockSpec((1, _D), lambda i: (0, 0)),
            pl.BlockSpec((_D, _D), lambda i: (0, 0)),
            pl.BlockSpec((1, _D), lambda i: (0, 0)),
        ],
        out_specs=[
            pl.BlockSpec((_BN, _D), lambda i: (i, 0)),
            pl.BlockSpec((8, _D), lambda i: (0, 0)),
        ],
        out_shape=[
            jax.ShapeDtypeStruct((_N, _D), jnp.float32),
            jax.ShapeDtypeStruct((8, _D), jnp.float32),
        ],
        scratch_shapes=[pltpu.VMEM((8, _D), jnp.float32)],
    )(a, b, Wa, ba, Wb, bb)


def _norm(v, sums, g, be):
    def body(v_ref, s_ref, g_ref, be_ref, h_ref):
        mu = s_ref[0:1, :] * (1.0 / _N)
        var = s_ref[1:2, :] * (1.0 / _N) - mu * mu
        scale = lax.rsqrt(var + 1e-5) * g_ref[...]
        h_ref[...] = (v_ref[...] - mu) * scale + be_ref[...]

    return pl.pallas_call(
        body,
        grid=(_N // _BN,),
        in_specs=[
            pl.BlockSpec((_BN, _D), lambda i: (i, 0)),
            pl.BlockSpec((8, _D), lambda i: (0, 0)),
            pl.BlockSpec((1, _D), lambda i: (0, 0)),
            pl.BlockSpec((1, _D), lambda i: (0, 0)),
        ],
        out_specs=pl.BlockSpec((_BN, _D), lambda i: (i, 0)),
        out_shape=jax.ShapeDtypeStruct((_N, _D), jnp.float32),
    )(v, sums, g, be)


def _final(v, sums, g, be, fc1_W, fc1_b, fc2_W, fc2_b):
    nc = fc2_W.shape[1]

    def body(v_ref, s_ref, g_ref, be_ref, w1_ref, b1_ref, w2_ref, b2_ref, o_ref):
        mu = s_ref[0:1, :] * (1.0 / _N)
        var = s_ref[1:2, :] * (1.0 / _N) - mu * mu
        scale = lax.rsqrt(var + 1e-5) * g_ref[...]
        h = (v_ref[...] - mu) * scale + be_ref[...]
        r = jnp.maximum(
            jnp.dot(h, w1_ref[...], preferred_element_type=jnp.float32) + b1_ref[...],
            0.0)
        o = jnp.dot(r, w2_ref[...], preferred_element_type=jnp.float32) + b2_ref[...]
        m = jnp.max(o, axis=-1, keepdims=True)
        lse = m + jnp.log(jnp.sum(jnp.exp(o - m), axis=-1, keepdims=True))
        o_ref[...] = o - lse

    return pl.pallas_call(
        body,
        grid=(_N // _BN,),
        in_specs=[
            pl.BlockSpec((_BN, _D), lambda i: (i, 0)),
            pl.BlockSpec((8, _D), lambda i: (0, 0)),
            pl.BlockSpec((1, _D), lambda i: (0, 0)),
            pl.BlockSpec((1, _D), lambda i: (0, 0)),
            pl.BlockSpec((_D, _D), lambda i: (0, 0)),
            pl.BlockSpec((1, _D), lambda i: (0, 0)),
            pl.BlockSpec((_D, nc), lambda i: (0, 0)),
            pl.BlockSpec((1, nc), lambda i: (0, 0)),
        ],
        out_specs=pl.BlockSpec((_BN, nc), lambda i: (i, 0)),
        out_shape=jax.ShapeDtypeStruct((_N, nc), jnp.float32),
    )(v, sums, g, be, fc1_W, fc1_b, fc2_W, fc2_b)


def kernel(x, edge_index, W1a, b1a, W1b, b1b, g1, be1, W2a, b2a, W2b, b2b, g2,
           be2, W3a, b3a, W3b, b3b, g3, be3, W4a, b4a, W4b, b4b, g4, be4, W5a,
           b5a, W5b, b5b, g5, be5, fc1_W, fc1_b, fc2_W, fc2_b):
    src = edge_index[0]
    dst = edge_index[1]
    eye = jnp.eye(_D, dtype=jnp.float32)

    y1 = _expand(x, W1a)
    aggy = _sc_scatter_add(y1, src, dst)
    v, sums = _mlp(y1, aggy, eye, b1a.reshape(1, _D), W1b, b1b.reshape(1, _D),
                   first_dot=False)
    h = _norm(v, sums, g1.reshape(1, _D), be1.reshape(1, _D))

    layers = [
        (W2a, b2a, W2b, b2b, g2, be2),
        (W3a, b3a, W3b, b3b, g3, be3),
        (W4a, b4a, W4b, b4b, g4, be4),
        (W5a, b5a, W5b, b5b, g5, be5),
    ]
    for li, (Wa, ba, Wb, bb, g, be) in enumerate(layers):
        agg = _sc_scatter_add(h, src, dst)
        v, sums = _mlp(h, agg, Wa, ba.reshape(1, _D), Wb, bb.reshape(1, _D))
        if li < len(layers) - 1:
            h = _norm(v, sums, g.reshape(1, _D), be.reshape(1, _D))

    return _final(v, sums, g5.reshape(1, _D), be5.reshape(1, _D), fc1_W,
                  fc1_b.reshape(1, _D), fc2_W, fc2_b.reshape(1, 2))
